# baseline (speedup 1.0000x reference)
import functools

import jax
import jax.numpy as jnp
from jax import lax
from jax.experimental import pallas as pl
from jax.experimental.pallas import tpu as pltpu
from jax.experimental.pallas import tpu_sc as plsc

_OC, _IC, _K = 192, 192, 3
_B, _H, _W = 8, 56, 56
_HP, _WP = _H + 2, _W + 2
_PFLAT = _HP * _WP
_NOUT = 3328
_XCOLS = 3456
_TOTAL = _OC * _IC * _K * _K
_NW = 32
_ROWS = _TOTAL // _NW
_SENTINEL = 1 << 30


def _make_weight_scatter(nnz_pad):
    mesh = plsc.VectorSubcoreMesh(core_axis_name="c", subcore_axis_name="s")

    @functools.partial(
        pl.kernel,
        out_type=jax.ShapeDtypeStruct((_TOTAL,), jnp.float32),
        mesh=mesh,
        compiler_params=pltpu.CompilerParams(needs_layout_passes=False),
        scratch_types=[
            pltpu.VMEM((nnz_pad,), jnp.int32),
            pltpu.VMEM((nnz_pad,), jnp.float32),
            pltpu.VMEM((_ROWS,), jnp.float32),
        ],
    )
    def weight_scatter(idx_hbm, val_hbm, out_hbm, idx_v, val_v, acc_v):
        wid = lax.axis_index("s") * 2 + lax.axis_index("c")
        base = wid * _ROWS
        pltpu.sync_copy(idx_hbm, idx_v)
        pltpu.sync_copy(val_hbm, val_v)

        def zero_body(i, carry):
            acc_v[pl.ds(pl.multiple_of(i * 16, 16), 16)] = jnp.zeros(
                (16,), jnp.float32)
            return carry

        lax.fori_loop(0, _ROWS // 16, zero_body, 0)

        def scan_body(i, carry):
            off = pl.multiple_of(i * 16, 16)
            idx = idx_v[pl.ds(off, 16)]
            val = val_v[pl.ds(off, 16)]
            loc = idx - base
            mask = (loc >= 0) & (loc < _ROWS)
            loc = jnp.where(mask, loc, 0)
            plsc.store_scatter(acc_v, [loc], val, mask=mask)
            return carry

        lax.fori_loop(0, nnz_pad // 16, scan_body, 0)
        pltpu.sync_copy(acc_v, out_hbm.at[pl.ds(base, _ROWS)])

    return weight_scatter


def _conv_body(x_ref, w_ref, b_ref, o_ref):
    acc = jnp.zeros((_OC, _NOUT), jnp.float32)
    for j in range(_K * _K):
        s = (j // _K) * _WP + (j % _K)
        acc = acc + lax.dot(
            w_ref[j],
            x_ref[0, :, s:s + _NOUT],
            precision=lax.Precision.HIGHEST,
            preferred_element_type=jnp.float32,
        )
    o_ref[0] = acc + b_ref[...]


def kernel(input, W_val, bias, W_idx):
    nnz = W_val.shape[0]
    oc, ic, kx, ky = W_idx[0], W_idx[1], W_idx[2], W_idx[3]
    flat = ((oc * _IC + ic) * _K + kx) * _K + ky
    nnz_pad = -(-nnz // 16) * 16
    pad = nnz_pad - nnz
    flat_p = jnp.concatenate(
        [flat, jnp.full((pad,), _SENTINEL, jnp.int32)])
    val_p = jnp.concatenate([W_val, jnp.zeros((pad,), jnp.float32)])

    dense_flat = _make_weight_scatter(nnz_pad)(flat_p, val_p)
    w9 = (dense_flat.reshape(_OC, _IC, _K, _K)
          .transpose(2, 3, 0, 1)
          .reshape(_K * _K, _OC, _IC))

    xp = jnp.pad(input, ((0, 0), (0, 0), (1, 1), (1, 1)))
    xflat = xp.reshape(_B, _IC, _PFLAT)
    xflat = jnp.pad(xflat, ((0, 0), (0, 0), (0, _XCOLS - _PFLAT)))

    out = pl.pallas_call(
        _conv_body,
        grid=(_B,),
        in_specs=[
            pl.BlockSpec((1, _IC, _XCOLS), lambda b: (b, 0, 0)),
            pl.BlockSpec((_K * _K, _OC, _IC), lambda b: (0, 0, 0)),
            pl.BlockSpec((_OC, 1), lambda b: (0, 0)),
        ],
        out_specs=pl.BlockSpec((1, _OC, _NOUT), lambda b: (b, 0, 0)),
        out_shape=jax.ShapeDtypeStruct((_B, _OC, _NOUT), jnp.float32),
    )(xflat, w9, bias.reshape(_OC, 1))

    out = out[:, :, :_H * _WP].reshape(_B, _OC, _H, _WP)[:, :, :, :_W]
    return out


# speedup vs baseline: 2.1801x; 2.1801x over previous
import functools

import jax
import jax.numpy as jnp
from jax import lax
from jax.experimental import pallas as pl
from jax.experimental.pallas import tpu as pltpu
from jax.experimental.pallas import tpu_sc as plsc

_OC, _IC, _K = 192, 192, 3
_B, _H, _W = 8, 56, 56
_STR = 128
_NOUT = _H * _STR
_XCOLS = 60 * _STR
_TOTAL = _OC * _IC * _K * _K
_NW = 32
_ROWS = _TOTAL // _NW
_OCPW = _OC // _NW
_JBLK = _OCPW * _IC
_SENTINEL = 1 << 30


def _make_weight_scatter(nnz_pad):
    mesh = plsc.VectorSubcoreMesh(core_axis_name="c", subcore_axis_name="s")

    @functools.partial(
        pl.kernel,
        out_type=jax.ShapeDtypeStruct((_TOTAL,), jnp.float32),
        mesh=mesh,
        compiler_params=pltpu.CompilerParams(needs_layout_passes=False),
        scratch_types=[
            pltpu.VMEM((nnz_pad,), jnp.int32),
            pltpu.VMEM((nnz_pad,), jnp.float32),
            pltpu.VMEM((48,), jnp.int32),
            pltpu.VMEM((_ROWS,), jnp.float32),
            pltpu.VMEM((_ROWS,), jnp.float32),
        ],
    )
    def weight_scatter(idx_hbm, val_hbm, bnd_hbm, out_hbm,
                       idx_v, val_v, bnd_v, acc_v, tr_v):
        wid = lax.axis_index("s") * 2 + lax.axis_index("c")
        base = wid * _ROWS
        pltpu.sync_copy(idx_hbm, idx_v)
        pltpu.sync_copy(val_hbm, val_v)
        pltpu.sync_copy(bnd_hbm, bnd_v)

        lanes = lax.iota(jnp.int32, 16)

        def read_bound(k):
            ch = k // 16
            ln = k % 16
            v0 = bnd_v[pl.ds(pl.multiple_of(ch * 16, 16), 16)]
            return jnp.sum(jnp.where(lanes == ln, v0, 0))

        lo = read_bound(wid)
        hi = read_bound(wid + 1)

        def zero_body(i, carry):
            acc_v[pl.ds(pl.multiple_of(i * 16, 16), 16)] = jnp.zeros(
                (16,), jnp.float32)
            return carry

        lax.fori_loop(0, _ROWS // 16, zero_body, 0)

        def scan_body(i, carry):
            off = pl.multiple_of(i * 16, 16)
            idx = idx_v[pl.ds(off, 16)]
            val = val_v[pl.ds(off, 16)]
            loc = idx - base
            mask = (loc >= 0) & (loc < _ROWS)
            loc = jnp.where(mask, loc, 0)
            plsc.store_scatter(acc_v, [loc], val, mask=mask)
            return carry

        lax.fori_loop(lo // 16, (hi + 15) // 16, scan_body, 0)

        def perm_body(v, carry):
            q = lanes + v * 16
            j = q // _JBLK
            rem = q % _JBLK
            ocl = rem // _IC
            ic = rem % _IC
            fidx = ocl * (_IC * _K * _K) + ic * (_K * _K) + j
            tr_v[pl.ds(pl.multiple_of(v * 16, 16), 16)] = plsc.load_gather(
                acc_v, [fidx])
            return carry

        lax.fori_loop(0, _ROWS // 16, perm_body, 0)

        for j in range(_K * _K):
            pltpu.sync_copy(
                tr_v.at[pl.ds(j * _JBLK, _JBLK)],
                out_hbm.at[pl.ds(j * (_OC * _IC) + wid * _JBLK, _JBLK)])

    return weight_scatter


def _conv_body(x_ref, w_ref, b_ref, o_ref, xflat):
    b = pl.program_id(0)

    @pl.when(b == 0)
    def _():
        xflat[...] = jnp.zeros((_IC, _XCOLS), jnp.float32)

    for h in range(_H):
        xflat[:, pl.ds(_STR * (h + 2), _W)] = x_ref[0, :, h, :]

    xv = xflat[...]
    xhi = pltpu.roll(xv, 1, 1)
    xlo = pltpu.roll(xv, _XCOLS - 1, 1)

    acc = b_ref[...] + jnp.zeros((_OC, _NOUT), jnp.float32)
    for kx in range(_K):
        base = _STR * (kx + 1)
        for ky, arr in ((0, xhi), (1, xv), (2, xlo)):
            acc = acc + lax.dot(
                w_ref[kx * _K + ky],
                arr[:, base:base + _NOUT],
                preferred_element_type=jnp.float32,
            )

    for h in range(_H):
        o_ref[0, :, h, :] = acc[:, _STR * h:_STR * h + _W]


def kernel(input, W_val, bias, W_idx):
    nnz = W_val.shape[0]
    oc, ic, kx, ky = W_idx[0], W_idx[1], W_idx[2], W_idx[3]
    flat = ((oc * _IC + ic) * _K + kx) * _K + ky
    nnz_pad = -(-nnz // 16) * 16
    pad = nnz_pad - nnz
    flat_p = jnp.concatenate(
        [flat, jnp.full((pad,), _SENTINEL, jnp.int32)])
    val_p = jnp.concatenate([W_val, jnp.zeros((pad,), jnp.float32)])
    bnd = jnp.searchsorted(
        flat_p, jnp.arange(_NW + 1, dtype=jnp.int32) * _ROWS).astype(jnp.int32)
    bnd = jnp.pad(bnd, (0, 48 - (_NW + 1)))

    dense_flat = _make_weight_scatter(nnz_pad)(flat_p, val_p, bnd)
    w9 = dense_flat.reshape(_K * _K, _OC, _IC)

    out = pl.pallas_call(
        _conv_body,
        grid=(_B,),
        in_specs=[
            pl.BlockSpec((1, _IC, _H, _W), lambda b: (b, 0, 0, 0)),
            pl.BlockSpec((_K * _K, _OC, _IC), lambda b: (0, 0, 0)),
            pl.BlockSpec((_OC, 1), lambda b: (0, 0)),
        ],
        out_specs=pl.BlockSpec((1, _OC, _H, _W), lambda b: (b, 0, 0, 0)),
        out_shape=jax.ShapeDtypeStruct((_B, _OC, _H, _W), jnp.float32),
        scratch_shapes=[
            pltpu.VMEM((_IC, _XCOLS), jnp.float32),
        ],
    )(input, w9, bias.reshape(_OC, 1))
    return out


# speedup vs baseline: 5.4221x; 2.4871x over previous
import functools

import jax
import jax.numpy as jnp
from jax import lax
from jax.experimental import pallas as pl
from jax.experimental.pallas import tpu as pltpu
from jax.experimental.pallas import tpu_sc as plsc

_OC, _IC, _K = 192, 192, 3
_B, _H, _W = 8, 56, 56
_RST = 64
_MOUT = _H * _RST
_XSROWS = 60 * _RST
_TOTAL = _OC * _IC * _K * _K
_NW = 32
_ROWS = _TOTAL // _NW
_OCPW = _OC // _NW
_JBLK = _OCPW * _IC
_SENTINEL = 1 << 30


def _make_weight_scatter(nnz_pad):
    mesh = plsc.VectorSubcoreMesh(core_axis_name="c", subcore_axis_name="s")

    @functools.partial(
        pl.kernel,
        out_type=jax.ShapeDtypeStruct((_TOTAL,), jnp.float32),
        mesh=mesh,
        compiler_params=pltpu.CompilerParams(needs_layout_passes=False),
        scratch_types=[
            pltpu.VMEM((nnz_pad,), jnp.int32),
            pltpu.VMEM((nnz_pad,), jnp.float32),
            pltpu.VMEM((48,), jnp.int32),
            pltpu.VMEM((_ROWS,), jnp.float32),
            pltpu.VMEM((_ROWS,), jnp.float32),
        ],
    )
    def weight_scatter(idx_hbm, val_hbm, bnd_hbm, out_hbm,
                       idx_v, val_v, bnd_v, acc_v, tr_v):
        wid = lax.axis_index("s") * 2 + lax.axis_index("c")
        base = wid * _ROWS
        pltpu.sync_copy(idx_hbm, idx_v)
        pltpu.sync_copy(val_hbm, val_v)
        pltpu.sync_copy(bnd_hbm, bnd_v)

        lanes = lax.iota(jnp.int32, 16)

        def read_bound(k):
            ch = k // 16
            ln = k % 16
            v0 = bnd_v[pl.ds(pl.multiple_of(ch * 16, 16), 16)]
            return jnp.sum(jnp.where(lanes == ln, v0, 0))

        lo = read_bound(wid)
        hi = read_bound(wid + 1)

        def zero_body(i, carry):
            acc_v[pl.ds(pl.multiple_of(i * 16, 16), 16)] = jnp.zeros(
                (16,), jnp.float32)
            return carry

        lax.fori_loop(0, _ROWS // 16, zero_body, 0)

        def scan_body(i, carry):
            off = pl.multiple_of(i * 16, 16)
            idx = idx_v[pl.ds(off, 16)]
            val = val_v[pl.ds(off, 16)]
            loc = idx - base
            mask = (loc >= 0) & (loc < _ROWS)
            loc = jnp.where(mask, loc, 0)
            plsc.store_scatter(acc_v, [loc], val, mask=mask)
            return carry

        lax.fori_loop(lo // 16, (hi + 15) // 16, scan_body, 0)

        def perm_body(v, carry):
            q = lanes + v * 16
            j = q // _JBLK
            rem = q % _JBLK
            ocl = rem // _IC
            ic = rem % _IC
            fidx = ocl * (_IC * _K * _K) + ic * (_K * _K) + j
            tr_v[pl.ds(pl.multiple_of(v * 16, 16), 16)] = plsc.load_gather(
                acc_v, [fidx])
            return carry

        lax.fori_loop(0, _ROWS // 16, perm_body, 0)

        for j in range(_K * _K):
            pltpu.sync_copy(
                tr_v.at[pl.ds(j * _JBLK, _JBLK)],
                out_hbm.at[pl.ds(j * (_OC * _IC) + wid * _JBLK, _JBLK)])

    return weight_scatter


def _conv_body(x_ref, w_ref, b_ref, o_ref, xs):
    b = pl.program_id(0)

    @pl.when(b == 0)
    def _():
        xs[...] = jnp.zeros((_XSROWS, _IC), jnp.float32)

    for r in range(_H):
        xs[pl.ds(_RST * (r + 2), _W), :] = x_ref[0, r]

    xv = xs[...]
    xup = pltpu.roll(xv, 1, 0)
    xdn = pltpu.roll(xv, _XSROWS - 1, 0)

    acc = b_ref[...] + jnp.zeros((_MOUT, _OC), jnp.float32)
    for kx in range(_K):
        base = _RST * (kx + 1)
        for ky, arr in ((0, xup), (1, xv), (2, xdn)):
            acc = acc + lax.dot_general(
                arr[base:base + _MOUT, :],
                w_ref[kx * _K + ky],
                (((1,), (1,)), ((), ())),
                preferred_element_type=jnp.float32,
            )

    for h in range(_H):
        o_ref[0, h] = acc[_RST * h:_RST * h + _W, :]


def kernel(input, W_val, bias, W_idx):
    nnz = W_val.shape[0]
    oc, ic, kx, ky = W_idx[0], W_idx[1], W_idx[2], W_idx[3]
    flat = ((oc * _IC + ic) * _K + kx) * _K + ky
    nnz_pad = -(-nnz // 16) * 16
    pad = nnz_pad - nnz
    flat_p = jnp.concatenate(
        [flat, jnp.full((pad,), _SENTINEL, jnp.int32)])
    val_p = jnp.concatenate([W_val, jnp.zeros((pad,), jnp.float32)])
    bnd = jnp.searchsorted(
        flat_p, jnp.arange(_NW + 1, dtype=jnp.int32) * _ROWS).astype(jnp.int32)
    bnd = jnp.pad(bnd, (0, 48 - (_NW + 1)))

    dense_flat = _make_weight_scatter(nnz_pad)(flat_p, val_p, bnd)
    w9 = dense_flat.reshape(_K * _K, _OC, _IC)

    x_nhwc = jnp.transpose(input, (0, 2, 3, 1))
    out = pl.pallas_call(
        _conv_body,
        grid=(_B,),
        in_specs=[
            pl.BlockSpec((1, _H, _W, _IC), lambda b: (b, 0, 0, 0)),
            pl.BlockSpec((_K * _K, _OC, _IC), lambda b: (0, 0, 0)),
            pl.BlockSpec((1, _OC), lambda b: (0, 0)),
        ],
        out_specs=pl.BlockSpec((1, _H, _W, _OC), lambda b: (b, 0, 0, 0)),
        out_shape=jax.ShapeDtypeStruct((_B, _H, _W, _OC), jnp.float32),
        scratch_shapes=[
            pltpu.VMEM((_XSROWS, _IC), jnp.float32),
        ],
    )(x_nhwc, w9, bias.reshape(1, _OC))
    return jnp.transpose(out, (0, 3, 1, 2))


# speedup vs baseline: 5.9959x; 1.1058x over previous
import functools

import jax
import jax.numpy as jnp
from jax import lax
from jax.experimental import pallas as pl
from jax.experimental.pallas import tpu as pltpu
from jax.experimental.pallas import tpu_sc as plsc

_OC, _IC, _K = 192, 192, 3
_B, _H, _W = 8, 56, 56
_RST = 64
_MOUT = _H * _RST
_XSROWS = 60 * _RST
_TOTAL = _OC * _IC * _K * _K
_NW = 32
_ROWS = _TOTAL // _NW
_OCPW = _OC // _NW
_JBLK = _OCPW * _IC
_SENTINEL = 1 << 30


def _make_weight_scatter(nnz_pad):
    mesh = plsc.VectorSubcoreMesh(core_axis_name="c", subcore_axis_name="s")

    @functools.partial(
        pl.kernel,
        out_type=jax.ShapeDtypeStruct((_TOTAL,), jnp.float32),
        mesh=mesh,
        compiler_params=pltpu.CompilerParams(needs_layout_passes=False),
        scratch_types=[
            pltpu.VMEM((nnz_pad,), jnp.int32),
            pltpu.VMEM((nnz_pad,), jnp.float32),
            pltpu.VMEM((48,), jnp.int32),
            pltpu.VMEM((_ROWS,), jnp.float32),
            pltpu.SemaphoreType.DMA,
        ],
    )
    def weight_scatter(idx_hbm, val_hbm, bnd_hbm, out_hbm,
                       idx_v, val_v, bnd_v, tr_v, sem):
        wid = lax.axis_index("s") * 2 + lax.axis_index("c")
        base = wid * _ROWS
        c1 = pltpu.async_copy(idx_hbm, idx_v, sem)
        c2 = pltpu.async_copy(val_hbm, val_v, sem)
        c3 = pltpu.async_copy(bnd_hbm, bnd_v, sem)

        def zero_body(i, carry):
            tr_v[pl.ds(pl.multiple_of(i * 16, 16), 16)] = jnp.zeros(
                (16,), jnp.float32)
            return carry

        lax.fori_loop(0, _ROWS // 16, zero_body, 0)
        c1.wait()
        c2.wait()
        c3.wait()

        lanes = lax.iota(jnp.int32, 16)

        def read_bound(k):
            ch = k // 16
            ln = k % 16
            v0 = bnd_v[pl.ds(pl.multiple_of(ch * 16, 16), 16)]
            return jnp.sum(jnp.where(lanes == ln, v0, 0))

        lo = read_bound(wid)
        hi = read_bound(wid + 1)

        def scan_body(i, carry):
            off = pl.multiple_of(i * 16, 16)
            idx = idx_v[pl.ds(off, 16)]
            val = val_v[pl.ds(off, 16)]
            loc = idx - base
            mask = (loc >= 0) & (loc < _ROWS)
            loc = jnp.where(mask, loc, 0)
            ocl = loc // (_IC * _K * _K)
            rem = loc - ocl * (_IC * _K * _K)
            ic = rem // (_K * _K)
            j = rem - ic * (_K * _K)
            tidx = j * _JBLK + ocl * _IC + ic
            plsc.store_scatter(tr_v, [tidx], val, mask=mask)
            return carry

        lax.fori_loop(lo // 16, (hi + 15) // 16, scan_body, 0)

        for j in range(_K * _K):
            pltpu.sync_copy(
                tr_v.at[pl.ds(j * _JBLK, _JBLK)],
                out_hbm.at[pl.ds(j * (_OC * _IC) + wid * _JBLK, _JBLK)])

    return weight_scatter


def _conv_body(x_ref, w_ref, b_ref, o_ref, xs, wb):
    b = pl.program_id(0)

    @pl.when(b == 0)
    def _():
        xs[...] = jnp.zeros((_XSROWS, _IC), jnp.bfloat16)
        wb[...] = w_ref[...].astype(jnp.bfloat16)

    for r in range(_H):
        xs[pl.ds(_RST * (r + 2), _W), :] = x_ref[0, r].astype(jnp.bfloat16)

    xv = xs[...]
    xup = pltpu.roll(xv, 1, 0)
    xdn = pltpu.roll(xv, _XSROWS - 1, 0)

    acc = b_ref[...] + jnp.zeros((_MOUT, _OC), jnp.float32)
    for kx in range(_K):
        base = _RST * (kx + 1)
        for ky, arr in ((0, xup), (1, xv), (2, xdn)):
            acc = acc + lax.dot_general(
                arr[base:base + _MOUT, :],
                wb[kx * _K + ky],
                (((1,), (1,)), ((), ())),
                preferred_element_type=jnp.float32,
            )

    for h in range(_H):
        o_ref[0, h] = acc[_RST * h:_RST * h + _W, :]


def kernel(input, W_val, bias, W_idx):
    nnz = W_val.shape[0]
    oc, ic, kx, ky = W_idx[0], W_idx[1], W_idx[2], W_idx[3]
    flat = ((oc * _IC + ic) * _K + kx) * _K + ky
    nnz_pad = -(-nnz // 16) * 16
    pad = nnz_pad - nnz
    flat_p = jnp.concatenate(
        [flat, jnp.full((pad,), _SENTINEL, jnp.int32)])
    val_p = jnp.concatenate([W_val, jnp.zeros((pad,), jnp.float32)])
    bases = jnp.arange(_NW + 1, dtype=jnp.int32) * _ROWS
    bnd = jnp.sum(flat_p[None, :] < bases[:, None], axis=1).astype(jnp.int32)
    bnd = jnp.pad(bnd, (0, 48 - (_NW + 1)))

    dense_flat = _make_weight_scatter(nnz_pad)(flat_p, val_p, bnd)
    w9 = dense_flat.reshape(_K * _K, _OC, _IC)

    x_nhwc = jnp.transpose(input, (0, 2, 3, 1))
    out = pl.pallas_call(
        _conv_body,
        grid=(_B,),
        in_specs=[
            pl.BlockSpec((1, _H, _W, _IC), lambda b: (b, 0, 0, 0)),
            pl.BlockSpec((_K * _K, _OC, _IC), lambda b: (0, 0, 0)),
            pl.BlockSpec((1, _OC), lambda b: (0, 0)),
        ],
        out_specs=pl.BlockSpec((1, _H, _W, _OC), lambda b: (b, 0, 0, 0)),
        out_shape=jax.ShapeDtypeStruct((_B, _H, _W, _OC), jnp.float32),
        scratch_shapes=[
            pltpu.VMEM((_XSROWS, _IC), jnp.bfloat16),
            pltpu.VMEM((_K * _K, _OC, _IC), jnp.bfloat16),
        ],
    )(x_nhwc, w9, bias.reshape(1, _OC))
    return jnp.transpose(out, (0, 3, 1, 2))


# speedup vs baseline: 6.3499x; 1.0590x over previous
import functools

import jax
import jax.numpy as jnp
from jax import lax
from jax.experimental import pallas as pl
from jax.experimental.pallas import tpu as pltpu
from jax.experimental.pallas import tpu_sc as plsc

_OC, _IC, _K = 192, 192, 3
_B, _H, _W = 8, 56, 56
_RST = 64
_MOUT = _H * _RST
_XSROWS = 60 * _RST
_TOTAL = _OC * _IC * _K * _K
_NW = 32
_ROWS = _TOTAL // _NW
_OCPW = _OC // _NW
_JBLK = _OCPW * _IC
_CH = 4096
_SENTINEL = 1 << 30


def _make_weight_scatter(nnz_pad):
    mesh = plsc.VectorSubcoreMesh(core_axis_name="c", subcore_axis_name="s")

    @functools.partial(
        pl.kernel,
        out_type=jax.ShapeDtypeStruct((_TOTAL,), jnp.float32),
        mesh=mesh,
        compiler_params=pltpu.CompilerParams(needs_layout_passes=False),
        scratch_types=[
            pltpu.VMEM((_CH,), jnp.int32),
            pltpu.VMEM((_CH,), jnp.float32),
            pltpu.VMEM((48,), jnp.int32),
            pltpu.VMEM((_ROWS,), jnp.float32),
            pltpu.SemaphoreType.DMA,
        ],
    )
    def weight_scatter(idx_hbm, val_hbm, bnd_hbm, out_hbm,
                       idx_v, val_v, bnd_v, tr_v, sem):
        wid = lax.axis_index("s") * 2 + lax.axis_index("c")
        base = wid * _ROWS
        c3 = pltpu.async_copy(bnd_hbm, bnd_v, sem)

        def zero_body(i, carry):
            tr_v[pl.ds(pl.multiple_of(i * 16, 16), 16)] = jnp.zeros(
                (16,), jnp.float32)
            return carry

        lax.fori_loop(0, _ROWS // 16, zero_body, 0)
        c3.wait()

        lanes = lax.iota(jnp.int32, 16)

        def read_bound(k):
            ch = k // 16
            ln = k % 16
            v0 = bnd_v[pl.ds(pl.multiple_of(ch * 16, 16), 16)]
            return jnp.sum(jnp.where(lanes == ln, v0, 0))

        lo = read_bound(wid)
        hi = read_bound(wid + 1)

        st0 = (lo // 8) * 8
        nc = (hi - st0 + _CH - 1) // _CH

        def chunk_body(c, carry):
            start = jnp.minimum(st0 + c * _CH, nnz_pad - _CH)
            ci = pltpu.async_copy(idx_hbm.at[pl.ds(start, _CH)], idx_v, sem)
            cv = pltpu.async_copy(val_hbm.at[pl.ds(start, _CH)], val_v, sem)
            ci.wait()
            cv.wait()

            def scan_body(i, carry2):
                off = pl.multiple_of(i * 16, 16)
                idx = idx_v[pl.ds(off, 16)]
                val = val_v[pl.ds(off, 16)]
                loc = idx - base
                mask = (loc >= 0) & (loc < _ROWS)
                loc = jnp.where(mask, loc, 0)
                ocl = loc // (_IC * _K * _K)
                rem = loc - ocl * (_IC * _K * _K)
                ic = rem // (_K * _K)
                j = rem - ic * (_K * _K)
                tidx = j * _JBLK + ocl * _IC + ic
                plsc.store_scatter(tr_v, [tidx], val, mask=mask)
                return carry2

            i0 = jnp.maximum(lo - start, 0) // 16
            i1 = jnp.minimum((hi - start + 15) // 16, _CH // 16)
            lax.fori_loop(i0, i1, scan_body, 0)
            return carry

        lax.fori_loop(0, nc, chunk_body, 0)

        for j in range(_K * _K):
            pltpu.sync_copy(
                tr_v.at[pl.ds(j * _JBLK, _JBLK)],
                out_hbm.at[pl.ds(j * (_OC * _IC) + wid * _JBLK, _JBLK)])

    return weight_scatter


def _conv_body(x_ref, w_ref, b_ref, o_ref, xs, wb):
    b = pl.program_id(0)

    @pl.when(b == 0)
    def _():
        xs[...] = jnp.zeros((_XSROWS, _IC), jnp.bfloat16)
        wb[...] = w_ref[...].astype(jnp.bfloat16)

    for r in range(_H):
        xs[pl.ds(_RST * (r + 2), _W), :] = x_ref[0, r].astype(jnp.bfloat16)

    xv = xs[...]
    xup = pltpu.roll(xv, 1, 0)
    xdn = pltpu.roll(xv, _XSROWS - 1, 0)

    acc = b_ref[...] + jnp.zeros((_MOUT, _OC), jnp.float32)
    for kx in range(_K):
        base = _RST * (kx + 1)
        for ky, arr in ((0, xup), (1, xv), (2, xdn)):
            acc = acc + lax.dot_general(
                arr[base:base + _MOUT, :],
                wb[kx * _K + ky],
                (((1,), (1,)), ((), ())),
                preferred_element_type=jnp.float32,
            )

    for h in range(_H):
        o_ref[0, h] = acc[_RST * h:_RST * h + _W, :]


def kernel(input, W_val, bias, W_idx):
    nnz = W_val.shape[0]
    oc, ic, kx, ky = W_idx[0], W_idx[1], W_idx[2], W_idx[3]
    flat = ((oc * _IC + ic) * _K + kx) * _K + ky
    nnz_pad = -(-nnz // 16) * 16
    pad = nnz_pad - nnz
    flat_p = jnp.concatenate(
        [flat, jnp.full((pad,), _SENTINEL, jnp.int32)])
    val_p = jnp.concatenate([W_val, jnp.zeros((pad,), jnp.float32)])
    bases = jnp.arange(_NW + 1, dtype=jnp.int32) * _ROWS
    bnd = jnp.sum(flat_p[None, :] < bases[:, None], axis=1).astype(jnp.int32)
    bnd = jnp.pad(bnd, (0, 48 - (_NW + 1)))

    dense_flat = _make_weight_scatter(nnz_pad)(flat_p, val_p, bnd)
    w9 = dense_flat.reshape(_K * _K, _OC, _IC)

    x_nhwc = jnp.transpose(input, (0, 2, 3, 1))
    out = pl.pallas_call(
        _conv_body,
        grid=(_B,),
        in_specs=[
            pl.BlockSpec((1, _H, _W, _IC), lambda b: (b, 0, 0, 0)),
            pl.BlockSpec((_K * _K, _OC, _IC), lambda b: (0, 0, 0)),
            pl.BlockSpec((1, _OC), lambda b: (0, 0)),
        ],
        out_specs=pl.BlockSpec((1, _H, _W, _OC), lambda b: (b, 0, 0, 0)),
        out_shape=jax.ShapeDtypeStruct((_B, _H, _W, _OC), jnp.float32),
        scratch_shapes=[
            pltpu.VMEM((_XSROWS, _IC), jnp.bfloat16),
            pltpu.VMEM((_K * _K, _OC, _IC), jnp.bfloat16),
        ],
    )(x_nhwc, w9, bias.reshape(1, _OC))
    return jnp.transpose(out, (0, 3, 1, 2))
